# baseline (speedup 1.0000x reference)
import functools

import jax
import jax.numpy as jnp
from jax import lax
from jax.experimental import pallas as pl
from jax.experimental.pallas import tpu as pltpu
from jax.experimental.pallas import tpu_sc as plsc

N = 10000
D = 128
E = 320000

NC = 2
NS = 16
H = 5120
N_PAD = NC * H
STRIPE = H // NS

K = 128
EPT = 20480
CHUNKS = EPT // K
E_PAD = NS * EPT
SENT = -1
NBUF = 2


@functools.cache
def _sc_kernels():
    mesh = plsc.VectorSubcoreMesh(
        core_axis_name="c", subcore_axis_name="s",
        num_cores=NC, num_subcores=NS)

    @functools.partial(
        pl.kernel,
        out_type=jax.ShapeDtypeStruct((N_PAD, D), jnp.float32),
        mesh=mesh,
        scratch_types=[
            pltpu.VMEM((CHUNKS, K), jnp.int32),
            pltpu.VMEM((CHUNKS, K), jnp.int32),
            [pltpu.VMEM((K, D), jnp.float32)] * NBUF,
            [pltpu.SemaphoreType.DMA] * NBUF,
            [pltpu.SemaphoreType.DMA] * NBUF,
            pltpu.VMEM_SHARED((H, D), jnp.float32),
        ],
    )
    def edge_kernel(h_hbm, srcg_hbm, dstd_hbm, zeros_hbm, out,
                    idx_s, idx_d, rows, gsem, ssem, agg):
        c = lax.axis_index("c")
        s = lax.axis_index("s")
        pltpu.sync_copy(zeros_hbm, agg.at[pl.ds(s * STRIPE, STRIPE)])
        pltpu.sync_copy(srcg_hbm.at[c, s], idx_s)
        pltpu.sync_copy(dstd_hbm.at[c, s], idx_d)
        plsc.subcore_barrier()

        def gidx(ci):
            return plsc.Indices(idx_s.at[ci], ignored_value=SENT)

        @pl.loop(0, CHUNKS, step=NBUF)
        def _(ci0):
            ds = [pltpu.async_copy(h_hbm.at[gidx(ci0 + b)], rows[b], gsem[b])
                  for b in range(NBUF)]
            for b in range(NBUF):
                ds[b].wait()
                pltpu.sync_copy(
                    rows[b],
                    agg.at[plsc.Indices(idx_d.at[ci0 + b],
                                        ignored_value=SENT)],
                    add=True)

        plsc.subcore_barrier()
        pltpu.sync_copy(agg.at[pl.ds(s * STRIPE, STRIPE)],
                        out.at[pl.ds(c * H + s * STRIPE, STRIPE)])

    return edge_kernel


BN = 256
NB = N_PAD // BN


def _elu(t):
    return jnp.where(t > 0, t, jnp.exp(jnp.minimum(t, 0.0)) - 1.0)


def _norm_body(degs_ref, degd_ref, ns_ref, nd_ref):
    ds_ = degs_ref[:, 0:1]
    dd_ = degd_ref[:, 0:1]
    ns = jnp.where(ds_ > 0, lax.rsqrt(jnp.maximum(ds_, 1.0)), 0.0)
    nd = jnp.where(dd_ > 0, lax.rsqrt(jnp.maximum(dd_, 1.0)), 0.0)
    ns_ref[...] = jnp.broadcast_to(ns, (BN, D))
    nd_ref[...] = jnp.broadcast_to(nd, (BN, D))


_norm_call = pl.pallas_call(
    _norm_body,
    grid=(NB,),
    in_specs=[
        pl.BlockSpec((BN, D), lambda i: (i, 0)),
        pl.BlockSpec((BN, D), lambda i: (i, 0)),
    ],
    out_specs=[
        pl.BlockSpec((BN, D), lambda i: (i, 0)),
        pl.BlockSpec((BN, D), lambda i: (i, 0)),
    ],
    out_shape=[
        jax.ShapeDtypeStruct((N_PAD, D), jnp.float32),
        jax.ShapeDtypeStruct((N_PAD, D), jnp.float32),
    ],
)


def _pre_body(x_ref, ns_ref, w_ref, o_ref):
    o_ref[...] = jnp.dot(x_ref[...] * ns_ref[...], w_ref[...],
                         preferred_element_type=jnp.float32)


_pre_call = pl.pallas_call(
    _pre_body,
    grid=(NB,),
    in_specs=[
        pl.BlockSpec((BN, D), lambda i: (i, 0)),
        pl.BlockSpec((BN, D), lambda i: (i, 0)),
        pl.BlockSpec((D, D), lambda i: (0, 0)),
    ],
    out_specs=pl.BlockSpec((BN, D), lambda i: (i, 0)),
    out_shape=jax.ShapeDtypeStruct((N_PAD, D), jnp.float32),
)


def _layer_body(p_ref, nd_ref, ns_ref, b_ref, w_ref, o_ref):
    t = _elu(p_ref[...] * nd_ref[...] + b_ref[...])
    o_ref[...] = jnp.dot(t * ns_ref[...], w_ref[...],
                         preferred_element_type=jnp.float32)


_layer_call = pl.pallas_call(
    _layer_body,
    grid=(NB,),
    in_specs=[
        pl.BlockSpec((BN, D), lambda i: (i, 0)),
        pl.BlockSpec((BN, D), lambda i: (i, 0)),
        pl.BlockSpec((BN, D), lambda i: (i, 0)),
        pl.BlockSpec((1, D), lambda i: (0, 0)),
        pl.BlockSpec((D, D), lambda i: (0, 0)),
    ],
    out_specs=pl.BlockSpec((BN, D), lambda i: (i, 0)),
    out_shape=jax.ShapeDtypeStruct((N_PAD, D), jnp.float32),
)


def _final_body(p_ref, nd_ref, b_ref, wl_ref, bl_ref, o_ref):
    t = _elu(p_ref[...] * nd_ref[...] + b_ref[...])
    o_ref[...] = jnp.dot(t, wl_ref[...],
                         preferred_element_type=jnp.float32) + bl_ref[...]


_final_call = pl.pallas_call(
    _final_body,
    grid=(NB,),
    in_specs=[
        pl.BlockSpec((BN, D), lambda i: (i, 0)),
        pl.BlockSpec((BN, D), lambda i: (i, 0)),
        pl.BlockSpec((1, D), lambda i: (0, 0)),
        pl.BlockSpec((D, D), lambda i: (0, 0)),
        pl.BlockSpec((1, D), lambda i: (0, 0)),
    ],
    out_specs=pl.BlockSpec((BN, D), lambda i: (i, 0)),
    out_shape=jax.ShapeDtypeStruct((N_PAD, D), jnp.float32),
)


def kernel(x, edge_index, W0, b0, W1, b1, W2, b2, Wl, bl):
    src = edge_index[0].astype(jnp.int32)
    dst = edge_index[1].astype(jnp.int32)
    npad = E_PAD - E
    src_g = jnp.concatenate([src, jnp.zeros((npad,), jnp.int32)])
    dst_p = jnp.concatenate([dst, jnp.full((npad,), SENT, jnp.int32)])
    src_p = jnp.concatenate([src, jnp.full((npad,), SENT, jnp.int32)])

    def remap(idx, c):
        lo, hi = c * H, (c + 1) * H
        ok = (idx >= lo) & (idx < hi)
        return jnp.where(ok, idx - lo, SENT)
    srcd_t = jnp.stack([remap(src_p, 0), remap(src_p, 1)]).reshape(
        NC, NS, CHUNKS, K)
    dstd_t = jnp.stack([remap(dst_p, 0), remap(dst_p, 1)]).reshape(
        NC, NS, CHUNKS, K)
    def gfilt(c):
        lo, hi = c * H, (c + 1) * H
        ok = (dst_p >= lo) & (dst_p < hi)
        return jnp.where(ok, src_g, SENT)
    srcg_t = jnp.stack([gfilt(0), gfilt(1)]).reshape(NC, NS, CHUNKS, K)

    zerosD = jnp.zeros((STRIPE, D), jnp.float32)

    edge_kernel = _sc_kernels()
    ones_tab = jnp.ones((N_PAD, D), jnp.float32)
    zerog_t = jnp.zeros((NC, NS, CHUNKS, K), jnp.int32)
    degs = edge_kernel(ones_tab, zerog_t, srcd_t, zerosD)
    degd = edge_kernel(ones_tab, zerog_t, dstd_t, zerosD)
    nsrc, ndst = _norm_call(degs, degd)

    x_pad = jnp.pad(x, ((0, N_PAD - N), (0, 0)))
    b0r = b0.reshape(1, D)
    b1r = b1.reshape(1, D)
    b2r = b2.reshape(1, D)
    blr = bl.reshape(1, D)

    h = _pre_call(x_pad, nsrc, W0)
    p = edge_kernel(h, srcg_t, dstd_t, zerosD)
    h = _layer_call(p, ndst, nsrc, b0r, W1)
    p = edge_kernel(h, srcg_t, dstd_t, zerosD)
    h = _layer_call(p, ndst, nsrc, b1r, W2)
    p = edge_kernel(h, srcg_t, dstd_t, zerosD)
    out = _final_call(p, ndst, b2r, Wl, blr)
    return out[:N]


# speedup vs baseline: 48.0696x; 48.0696x over previous
import functools

import jax
import jax.numpy as jnp
from jax import lax
from jax.experimental import pallas as pl
from jax.experimental.pallas import tpu as pltpu
from jax.experimental.pallas import tpu_sc as plsc

N = 10000
D = 128
E = 320000

NC = 2
NS = 16
H = 5120
N_PAD = NC * H
STRIPE = H // NS

K = 128
EPT = 20480
CHUNKS = EPT // K
E_PAD = NS * EPT
SENT = -1
NBUF = 2


@functools.cache
def _sc_kernels():
    mesh = plsc.VectorSubcoreMesh(
        core_axis_name="c", subcore_axis_name="s",
        num_cores=NC, num_subcores=NS)

    @functools.partial(
        pl.kernel,
        out_type=jax.ShapeDtypeStruct((N_PAD, D), jnp.float32),
        mesh=mesh,
        scratch_types=[
            pltpu.VMEM((CHUNKS, K), jnp.int32),
            pltpu.VMEM((CHUNKS, K), jnp.int32),
            pltpu.VMEM((K, D), jnp.float32),
            pltpu.VMEM((K, D), jnp.float32),
            pltpu.SemaphoreType.DMA,
            pltpu.SemaphoreType.DMA,
            pltpu.VMEM_SHARED((H, D), jnp.float32),
        ],
    )
    def deg_kernel(srcd_hbm, dstd_hbm, e0_hbm, e1_hbm, zeros_hbm, out,
                   idx_s, idx_d, e0_v, e1_v, sem0, sem1, hist):
        c = lax.axis_index("c")
        s = lax.axis_index("s")
        pltpu.sync_copy(zeros_hbm, hist.at[pl.ds(s * STRIPE, STRIPE)])
        pltpu.sync_copy(srcd_hbm.at[c, s], idx_s)
        pltpu.sync_copy(dstd_hbm.at[c, s], idx_d)
        pltpu.sync_copy(e0_hbm, e0_v)
        pltpu.sync_copy(e1_hbm, e1_v)
        plsc.subcore_barrier()

        @pl.loop(0, CHUNKS)
        def _(ci):
            d0 = pltpu.async_copy(
                e0_v,
                hist.at[plsc.Indices(idx_s.at[ci], ignored_value=SENT)],
                sem0, add=True)
            d1 = pltpu.async_copy(
                e1_v,
                hist.at[plsc.Indices(idx_d.at[ci], ignored_value=SENT)],
                sem1, add=True)
            d0.wait()
            d1.wait()

        plsc.subcore_barrier()
        pltpu.sync_copy(hist.at[pl.ds(s * STRIPE, STRIPE)],
                        out.at[pl.ds(c * H + s * STRIPE, STRIPE)])

    @functools.partial(
        pl.kernel,
        out_type=jax.ShapeDtypeStruct((N_PAD, D), jnp.float32),
        mesh=mesh,
        scratch_types=[
            pltpu.VMEM((CHUNKS, K), jnp.int32),
            pltpu.VMEM((CHUNKS, K), jnp.int32),
            [pltpu.VMEM((K, D), jnp.float32)] * NBUF,
            [pltpu.SemaphoreType.DMA] * NBUF,
            [pltpu.SemaphoreType.DMA] * NBUF,
            pltpu.VMEM_SHARED((H, D), jnp.float32),
        ],
    )
    def edge_kernel(h_hbm, srcg_hbm, dstd_hbm, zeros_hbm, out,
                    idx_s, idx_d, rows, gsem, ssem, agg):
        c = lax.axis_index("c")
        s = lax.axis_index("s")
        pltpu.sync_copy(zeros_hbm, agg.at[pl.ds(s * STRIPE, STRIPE)])
        pltpu.sync_copy(srcg_hbm.at[c, s], idx_s)
        pltpu.sync_copy(dstd_hbm.at[c, s], idx_d)
        plsc.subcore_barrier()

        def gidx(ci):
            return plsc.Indices(idx_s.at[ci], ignored_value=SENT)

        @pl.loop(0, CHUNKS, step=NBUF)
        def _(ci0):
            ds = [pltpu.async_copy(h_hbm.at[gidx(ci0 + b)], rows[b], gsem[b])
                  for b in range(NBUF)]
            for b in range(NBUF):
                ds[b].wait()
                pltpu.sync_copy(
                    rows[b],
                    agg.at[plsc.Indices(idx_d.at[ci0 + b],
                                        ignored_value=SENT)],
                    add=True)

        plsc.subcore_barrier()
        pltpu.sync_copy(agg.at[pl.ds(s * STRIPE, STRIPE)],
                        out.at[pl.ds(c * H + s * STRIPE, STRIPE)])

    return deg_kernel, edge_kernel


BN = 256
NB = N_PAD // BN


def _elu(t):
    return jnp.where(t > 0, t, jnp.exp(jnp.minimum(t, 0.0)) - 1.0)


def _norm_body(deg_ref, ns_ref, nd_ref):
    ds_ = deg_ref[:, 0:1]
    dd_ = deg_ref[:, 1:2]
    ns = jnp.where(ds_ > 0, lax.rsqrt(jnp.maximum(ds_, 1.0)), 0.0)
    nd = jnp.where(dd_ > 0, lax.rsqrt(jnp.maximum(dd_, 1.0)), 0.0)
    ns_ref[...] = jnp.broadcast_to(ns, (BN, D))
    nd_ref[...] = jnp.broadcast_to(nd, (BN, D))


_norm_call = pl.pallas_call(
    _norm_body,
    grid=(NB,),
    in_specs=[
        pl.BlockSpec((BN, D), lambda i: (i, 0)),
    ],
    out_specs=[
        pl.BlockSpec((BN, D), lambda i: (i, 0)),
        pl.BlockSpec((BN, D), lambda i: (i, 0)),
    ],
    out_shape=[
        jax.ShapeDtypeStruct((N_PAD, D), jnp.float32),
        jax.ShapeDtypeStruct((N_PAD, D), jnp.float32),
    ],
)


def _pre_body(x_ref, ns_ref, w_ref, o_ref):
    o_ref[...] = jnp.dot(x_ref[...] * ns_ref[...], w_ref[...],
                         preferred_element_type=jnp.float32)


_pre_call = pl.pallas_call(
    _pre_body,
    grid=(NB,),
    in_specs=[
        pl.BlockSpec((BN, D), lambda i: (i, 0)),
        pl.BlockSpec((BN, D), lambda i: (i, 0)),
        pl.BlockSpec((D, D), lambda i: (0, 0)),
    ],
    out_specs=pl.BlockSpec((BN, D), lambda i: (i, 0)),
    out_shape=jax.ShapeDtypeStruct((N_PAD, D), jnp.float32),
)


def _layer_body(p_ref, nd_ref, ns_ref, b_ref, w_ref, o_ref):
    t = _elu(p_ref[...] * nd_ref[...] + b_ref[...])
    o_ref[...] = jnp.dot(t * ns_ref[...], w_ref[...],
                         preferred_element_type=jnp.float32)


_layer_call = pl.pallas_call(
    _layer_body,
    grid=(NB,),
    in_specs=[
        pl.BlockSpec((BN, D), lambda i: (i, 0)),
        pl.BlockSpec((BN, D), lambda i: (i, 0)),
        pl.BlockSpec((BN, D), lambda i: (i, 0)),
        pl.BlockSpec((1, D), lambda i: (0, 0)),
        pl.BlockSpec((D, D), lambda i: (0, 0)),
    ],
    out_specs=pl.BlockSpec((BN, D), lambda i: (i, 0)),
    out_shape=jax.ShapeDtypeStruct((N_PAD, D), jnp.float32),
)


def _final_body(p_ref, nd_ref, b_ref, wl_ref, bl_ref, o_ref):
    t = _elu(p_ref[...] * nd_ref[...] + b_ref[...])
    o_ref[...] = jnp.dot(t, wl_ref[...],
                         preferred_element_type=jnp.float32) + bl_ref[...]


_final_call = pl.pallas_call(
    _final_body,
    grid=(NB,),
    in_specs=[
        pl.BlockSpec((BN, D), lambda i: (i, 0)),
        pl.BlockSpec((BN, D), lambda i: (i, 0)),
        pl.BlockSpec((1, D), lambda i: (0, 0)),
        pl.BlockSpec((D, D), lambda i: (0, 0)),
        pl.BlockSpec((1, D), lambda i: (0, 0)),
    ],
    out_specs=pl.BlockSpec((BN, D), lambda i: (i, 0)),
    out_shape=jax.ShapeDtypeStruct((N_PAD, D), jnp.float32),
)


def kernel(x, edge_index, W0, b0, W1, b1, W2, b2, Wl, bl):
    src = edge_index[0].astype(jnp.int32)
    dst = edge_index[1].astype(jnp.int32)
    npad = E_PAD - E
    src_g = jnp.concatenate([src, jnp.zeros((npad,), jnp.int32)])
    dst_p = jnp.concatenate([dst, jnp.full((npad,), SENT, jnp.int32)])
    src_p = jnp.concatenate([src, jnp.full((npad,), SENT, jnp.int32)])

    def remap(idx, c):
        lo, hi = c * H, (c + 1) * H
        ok = (idx >= lo) & (idx < hi)
        return jnp.where(ok, idx - lo, SENT)
    srcd_t = jnp.stack([remap(src_p, 0), remap(src_p, 1)]).reshape(
        NC, NS, CHUNKS, K)
    dstd_t = jnp.stack([remap(dst_p, 0), remap(dst_p, 1)]).reshape(
        NC, NS, CHUNKS, K)
    def gfilt(c):
        lo, hi = c * H, (c + 1) * H
        ok = (dst_p >= lo) & (dst_p < hi)
        return jnp.where(ok, src_g, SENT)
    srcg_t = jnp.stack([gfilt(0), gfilt(1)]).reshape(NC, NS, CHUNKS, K)

    zerosD = jnp.zeros((STRIPE, D), jnp.float32)

    deg_kernel, edge_kernel = _sc_kernels()
    col = jnp.arange(D, dtype=jnp.int32)
    e0 = jnp.broadcast_to((col == 0).astype(jnp.float32), (K, D))
    e1 = jnp.broadcast_to((col == 1).astype(jnp.float32), (K, D))
    deg = deg_kernel(srcd_t, dstd_t, e0, e1, zerosD)
    nsrc, ndst = _norm_call(deg)

    x_pad = jnp.pad(x, ((0, N_PAD - N), (0, 0)))
    b0r = b0.reshape(1, D)
    b1r = b1.reshape(1, D)
    b2r = b2.reshape(1, D)
    blr = bl.reshape(1, D)

    h = _pre_call(x_pad, nsrc, W0)
    p = edge_kernel(h, srcg_t, dstd_t, zerosD)
    h = _layer_call(p, ndst, nsrc, b0r, W1)
    p = edge_kernel(h, srcg_t, dstd_t, zerosD)
    h = _layer_call(p, ndst, nsrc, b1r, W2)
    p = edge_kernel(h, srcg_t, dstd_t, zerosD)
    out = _final_call(p, ndst, b2r, Wl, blr)
    return out[:N]


# speedup vs baseline: 49.2559x; 1.0247x over previous
import functools

import jax
import jax.numpy as jnp
from jax import lax
from jax.experimental import pallas as pl
from jax.experimental.pallas import tpu as pltpu
from jax.experimental.pallas import tpu_sc as plsc

N = 10000
D = 128
E = 320000

NC = 2
NS = 16
H = 5120
N_PAD = NC * H
STRIPE = H // NS

K = 128
EPT = 20480
CHUNKS = EPT // K
E_PAD = NS * EPT
SENT = -1
NBUF = 2


@functools.cache
def _sc_kernels():
    mesh = plsc.VectorSubcoreMesh(
        core_axis_name="c", subcore_axis_name="s",
        num_cores=NC, num_subcores=NS)

    @functools.partial(
        pl.kernel,
        out_type=jax.ShapeDtypeStruct((N_PAD, D), jnp.float32),
        mesh=mesh,
        scratch_types=[
            pltpu.VMEM((CHUNKS, K), jnp.int32),
            pltpu.VMEM((CHUNKS, K), jnp.int32),
            pltpu.VMEM((K, D), jnp.float32),
            pltpu.VMEM((K, D), jnp.float32),
            pltpu.SemaphoreType.DMA,
            pltpu.SemaphoreType.DMA,
            pltpu.VMEM_SHARED((H, D), jnp.float32),
        ],
    )
    def deg_kernel(srcd_hbm, dstd_hbm, e0_hbm, e1_hbm, zeros_hbm, out,
                   idx_s, idx_d, e0_v, e1_v, sem0, sem1, hist):
        c = lax.axis_index("c")
        s = lax.axis_index("s")
        pltpu.sync_copy(zeros_hbm, hist.at[pl.ds(s * STRIPE, STRIPE)])
        pltpu.sync_copy(srcd_hbm.at[c, s], idx_s)
        pltpu.sync_copy(dstd_hbm.at[c, s], idx_d)
        pltpu.sync_copy(e0_hbm, e0_v)
        pltpu.sync_copy(e1_hbm, e1_v)
        plsc.subcore_barrier()

        @pl.loop(0, CHUNKS)
        def _(ci):
            d0 = pltpu.async_copy(
                e0_v,
                hist.at[plsc.Indices(idx_s.at[ci], ignored_value=SENT)],
                sem0, add=True)
            d1 = pltpu.async_copy(
                e1_v,
                hist.at[plsc.Indices(idx_d.at[ci], ignored_value=SENT)],
                sem1, add=True)
            d0.wait()
            d1.wait()

        plsc.subcore_barrier()
        pltpu.sync_copy(hist.at[pl.ds(s * STRIPE, STRIPE)],
                        out.at[pl.ds(c * H + s * STRIPE, STRIPE)])

    @functools.partial(
        pl.kernel,
        out_type=jax.ShapeDtypeStruct((N_PAD, D), jnp.float32),
        mesh=mesh,
        scratch_types=[
            pltpu.VMEM((CHUNKS, K), jnp.int32),
            pltpu.VMEM((CHUNKS, K), jnp.int32),
            [pltpu.VMEM((K, D), jnp.float32)] * NBUF,
            [pltpu.SemaphoreType.DMA] * NBUF,
            [pltpu.SemaphoreType.DMA] * NBUF,
            pltpu.VMEM_SHARED((H, D), jnp.float32),
        ],
    )
    def edge_kernel(h_hbm, srcg_hbm, dstd_hbm, zeros_hbm, out,
                    idx_s, idx_d, rows, gsem, ssem, agg):
        c = lax.axis_index("c")
        s = lax.axis_index("s")
        pltpu.sync_copy(zeros_hbm, agg.at[pl.ds(s * STRIPE, STRIPE)])
        pltpu.sync_copy(srcg_hbm.at[c, s], idx_s)
        pltpu.sync_copy(dstd_hbm.at[c, s], idx_d)
        plsc.subcore_barrier()

        def gidx(ci):
            return plsc.Indices(idx_s.at[ci], ignored_value=SENT)

        @pl.loop(0, CHUNKS, step=NBUF)
        def _(ci0):
            ds = [pltpu.async_copy(h_hbm.at[gidx(ci0 + b)], rows[b], gsem[b])
                  for b in range(NBUF)]
            for b in range(NBUF):
                ds[b].wait()
                pltpu.sync_copy(
                    rows[b],
                    agg.at[plsc.Indices(idx_d.at[ci0 + b],
                                        ignored_value=SENT)],
                    add=True)

        plsc.subcore_barrier()
        pltpu.sync_copy(agg.at[pl.ds(s * STRIPE, STRIPE)],
                        out.at[pl.ds(c * H + s * STRIPE, STRIPE)])

    return deg_kernel, edge_kernel


BN = 256
NB = N_PAD // BN


def _elu(t):
    return jnp.where(t > 0, t, jnp.exp(jnp.minimum(t, 0.0)) - 1.0)


def _norms(deg_ref):
    ds_ = deg_ref[:, 0:1]
    dd_ = deg_ref[:, 1:2]
    ns = jnp.where(ds_ > 0, lax.rsqrt(jnp.maximum(ds_, 1.0)), 0.0)
    nd = jnp.where(dd_ > 0, lax.rsqrt(jnp.maximum(dd_, 1.0)), 0.0)
    return ns, nd


def _pre_body(x_ref, w_ref, o_ref):
    o_ref[...] = jnp.dot(x_ref[...], w_ref[...],
                         preferred_element_type=jnp.float32)


_pre_call = pl.pallas_call(
    _pre_body,
    grid=(NB,),
    in_specs=[
        pl.BlockSpec((BN, D), lambda i: (i, 0)),
        pl.BlockSpec((D, D), lambda i: (0, 0)),
    ],
    out_specs=pl.BlockSpec((BN, D), lambda i: (i, 0)),
    out_shape=jax.ShapeDtypeStruct((N_PAD, D), jnp.float32),
)


def _h0_body(xw_ref, deg_ref, o_ref):
    ns, _ = _norms(deg_ref)
    o_ref[...] = xw_ref[...] * ns


_h0_call = pl.pallas_call(
    _h0_body,
    grid=(NB,),
    in_specs=[
        pl.BlockSpec((BN, D), lambda i: (i, 0)),
        pl.BlockSpec((BN, D), lambda i: (i, 0)),
    ],
    out_specs=pl.BlockSpec((BN, D), lambda i: (i, 0)),
    out_shape=jax.ShapeDtypeStruct((N_PAD, D), jnp.float32),
)


def _layer_body(p_ref, deg_ref, b_ref, w_ref, o_ref):
    ns, nd = _norms(deg_ref)
    t = _elu(p_ref[...] * nd + b_ref[...])
    o_ref[...] = jnp.dot(t * ns, w_ref[...],
                         preferred_element_type=jnp.float32)


_layer_call = pl.pallas_call(
    _layer_body,
    grid=(NB,),
    in_specs=[
        pl.BlockSpec((BN, D), lambda i: (i, 0)),
        pl.BlockSpec((BN, D), lambda i: (i, 0)),
        pl.BlockSpec((1, D), lambda i: (0, 0)),
        pl.BlockSpec((D, D), lambda i: (0, 0)),
    ],
    out_specs=pl.BlockSpec((BN, D), lambda i: (i, 0)),
    out_shape=jax.ShapeDtypeStruct((N_PAD, D), jnp.float32),
)


def _final_body(p_ref, deg_ref, b_ref, wl_ref, bl_ref, o_ref):
    _, nd = _norms(deg_ref)
    t = _elu(p_ref[...] * nd + b_ref[...])
    o_ref[...] = jnp.dot(t, wl_ref[...],
                         preferred_element_type=jnp.float32) + bl_ref[...]


_final_call = pl.pallas_call(
    _final_body,
    grid=(NB,),
    in_specs=[
        pl.BlockSpec((BN, D), lambda i: (i, 0)),
        pl.BlockSpec((BN, D), lambda i: (i, 0)),
        pl.BlockSpec((1, D), lambda i: (0, 0)),
        pl.BlockSpec((D, D), lambda i: (0, 0)),
        pl.BlockSpec((1, D), lambda i: (0, 0)),
    ],
    out_specs=pl.BlockSpec((BN, D), lambda i: (i, 0)),
    out_shape=jax.ShapeDtypeStruct((N_PAD, D), jnp.float32),
)


def kernel(x, edge_index, W0, b0, W1, b1, W2, b2, Wl, bl):
    src = edge_index[0].astype(jnp.int32)
    dst = edge_index[1].astype(jnp.int32)
    npad = E_PAD - E
    src_g = jnp.concatenate([src, jnp.zeros((npad,), jnp.int32)])
    dst_p = jnp.concatenate([dst, jnp.full((npad,), SENT, jnp.int32)])
    src_p = jnp.concatenate([src, jnp.full((npad,), SENT, jnp.int32)])

    def remap(idx, c):
        lo, hi = c * H, (c + 1) * H
        ok = (idx >= lo) & (idx < hi)
        return jnp.where(ok, idx - lo, SENT)
    srcd_t = jnp.stack([remap(src_p, 0), remap(src_p, 1)]).reshape(
        NC, NS, CHUNKS, K)
    dstd_t = jnp.stack([remap(dst_p, 0), remap(dst_p, 1)]).reshape(
        NC, NS, CHUNKS, K)
    def gfilt(c):
        lo, hi = c * H, (c + 1) * H
        ok = (dst_p >= lo) & (dst_p < hi)
        return jnp.where(ok, src_g, SENT)
    srcg_t = jnp.stack([gfilt(0), gfilt(1)]).reshape(NC, NS, CHUNKS, K)

    zerosD = jnp.zeros((STRIPE, D), jnp.float32)

    deg_kernel, edge_kernel = _sc_kernels()
    col = jnp.arange(D, dtype=jnp.int32)
    e0 = jnp.broadcast_to((col == 0).astype(jnp.float32), (K, D))
    e1 = jnp.broadcast_to((col == 1).astype(jnp.float32), (K, D))
    deg = deg_kernel(srcd_t, dstd_t, e0, e1, zerosD)

    x_pad = jnp.pad(x, ((0, N_PAD - N), (0, 0)))
    b0r = b0.reshape(1, D)
    b1r = b1.reshape(1, D)
    b2r = b2.reshape(1, D)
    blr = bl.reshape(1, D)

    xw0 = _pre_call(x_pad, W0)
    h = _h0_call(xw0, deg)
    p = edge_kernel(h, srcg_t, dstd_t, zerosD)
    h = _layer_call(p, deg, b0r, W1)
    p = edge_kernel(h, srcg_t, dstd_t, zerosD)
    h = _layer_call(p, deg, b1r, W2)
    p = edge_kernel(h, srcg_t, dstd_t, zerosD)
    out = _final_call(p, deg, b2r, Wl, blr)
    return out[:N]


# speedup vs baseline: 50.1666x; 1.0185x over previous
import functools

import jax
import jax.numpy as jnp
from jax import lax
from jax.experimental import pallas as pl
from jax.experimental.pallas import tpu as pltpu
from jax.experimental.pallas import tpu_sc as plsc

N = 10000
D = 128
E = 320000

NC = 2
NS = 16
H = 5120
N_PAD = NC * H
STRIPE = H // NS

K = 128
EPT = 20480
CHUNKS = EPT // K
E_PAD = NS * EPT
SENT = -1
NBUF = 4
IB = 8


@functools.cache
def _sc_kernels():
    mesh = plsc.VectorSubcoreMesh(
        core_axis_name="c", subcore_axis_name="s",
        num_cores=NC, num_subcores=NS)

    @functools.partial(
        pl.kernel,
        out_type=jax.ShapeDtypeStruct((N_PAD, D), jnp.float32),
        mesh=mesh,
        scratch_types=[
            pltpu.VMEM((CHUNKS, K), jnp.int32),
            pltpu.VMEM((CHUNKS, K), jnp.int32),
            pltpu.VMEM((K, D), jnp.float32),
            pltpu.VMEM((K, D), jnp.float32),
            pltpu.SemaphoreType.DMA,
            pltpu.SemaphoreType.DMA,
            pltpu.VMEM_SHARED((H, D), jnp.float32),
        ],
    )
    def deg_kernel(srcd_hbm, dstd_hbm, e0_hbm, e1_hbm, zeros_hbm, out,
                   idx_s, idx_d, e0_v, e1_v, sem0, sem1, hist):
        c = lax.axis_index("c")
        s = lax.axis_index("s")
        pltpu.sync_copy(zeros_hbm, hist.at[pl.ds(s * STRIPE, STRIPE)])
        pltpu.sync_copy(srcd_hbm.at[c, s], idx_s)
        pltpu.sync_copy(dstd_hbm.at[c, s], idx_d)
        pltpu.sync_copy(e0_hbm, e0_v)
        pltpu.sync_copy(e1_hbm, e1_v)
        plsc.subcore_barrier()

        @pl.loop(0, CHUNKS)
        def _(ci):
            d0 = pltpu.async_copy(
                e0_v,
                hist.at[plsc.Indices(idx_s.at[ci], ignored_value=SENT)],
                sem0, add=True)
            d1 = pltpu.async_copy(
                e1_v,
                hist.at[plsc.Indices(idx_d.at[ci], ignored_value=SENT)],
                sem1, add=True)
            d0.wait()
            d1.wait()

        plsc.subcore_barrier()
        pltpu.sync_copy(hist.at[pl.ds(s * STRIPE, STRIPE)],
                        out.at[pl.ds(c * H + s * STRIPE, STRIPE)])

    @functools.partial(
        pl.kernel,
        out_type=jax.ShapeDtypeStruct((N_PAD, D), jnp.float32),
        mesh=mesh,
        scratch_types=[
            pltpu.VMEM((IB, K), jnp.int32),
            pltpu.VMEM((IB, K), jnp.int32),
            [pltpu.VMEM((K, D), jnp.float32)] * NBUF,
            [pltpu.SemaphoreType.DMA] * NBUF,
            pltpu.VMEM_SHARED((H, D), jnp.float32),
        ],
    )
    def edge_kernel(h_hbm, srcg_hbm, dstd_hbm, zeros_hbm, out,
                    idx_s, idx_d, rows, gsem, agg):
        c = lax.axis_index("c")
        s = lax.axis_index("s")
        pltpu.sync_copy(zeros_hbm, agg.at[pl.ds(s * STRIPE, STRIPE)])
        plsc.subcore_barrier()

        @pl.loop(0, CHUNKS, step=IB)
        def _(cb):
            pltpu.sync_copy(srcg_hbm.at[c, s, pl.ds(cb, IB)], idx_s)
            pltpu.sync_copy(dstd_hbm.at[c, s, pl.ds(cb, IB)], idx_d)
            for g in range(IB // NBUF):
                ds = [pltpu.async_copy(
                          h_hbm.at[plsc.Indices(idx_s.at[g * NBUF + b],
                                                ignored_value=SENT)],
                          rows[b], gsem[b])
                      for b in range(NBUF)]
                for b in range(NBUF):
                    ds[b].wait()
                    pltpu.sync_copy(
                        rows[b],
                        agg.at[plsc.Indices(idx_d.at[g * NBUF + b],
                                            ignored_value=SENT)],
                        add=True)

        plsc.subcore_barrier()
        pltpu.sync_copy(agg.at[pl.ds(s * STRIPE, STRIPE)],
                        out.at[pl.ds(c * H + s * STRIPE, STRIPE)])

    return deg_kernel, edge_kernel


BN = 256
NB = N_PAD // BN


def _elu(t):
    return jnp.where(t > 0, t, jnp.exp(jnp.minimum(t, 0.0)) - 1.0)


def _norms(deg_ref):
    ds_ = deg_ref[:, 0:1]
    dd_ = deg_ref[:, 1:2]
    ns = jnp.where(ds_ > 0, lax.rsqrt(jnp.maximum(ds_, 1.0)), 0.0)
    nd = jnp.where(dd_ > 0, lax.rsqrt(jnp.maximum(dd_, 1.0)), 0.0)
    return ns, nd


def _pre_body(x_ref, w_ref, o_ref):
    o_ref[...] = jnp.dot(x_ref[...], w_ref[...],
                         preferred_element_type=jnp.float32)


_pre_call = pl.pallas_call(
    _pre_body,
    grid=(NB,),
    in_specs=[
        pl.BlockSpec((BN, D), lambda i: (i, 0)),
        pl.BlockSpec((D, D), lambda i: (0, 0)),
    ],
    out_specs=pl.BlockSpec((BN, D), lambda i: (i, 0)),
    out_shape=jax.ShapeDtypeStruct((N_PAD, D), jnp.float32),
)


def _h0_body(xw_ref, deg_ref, o_ref):
    ns, _ = _norms(deg_ref)
    o_ref[...] = xw_ref[...] * ns


_h0_call = pl.pallas_call(
    _h0_body,
    grid=(NB,),
    in_specs=[
        pl.BlockSpec((BN, D), lambda i: (i, 0)),
        pl.BlockSpec((BN, D), lambda i: (i, 0)),
    ],
    out_specs=pl.BlockSpec((BN, D), lambda i: (i, 0)),
    out_shape=jax.ShapeDtypeStruct((N_PAD, D), jnp.float32),
)


def _layer_body(p_ref, deg_ref, b_ref, w_ref, o_ref):
    ns, nd = _norms(deg_ref)
    t = _elu(p_ref[...] * nd + b_ref[...])
    o_ref[...] = jnp.dot(t * ns, w_ref[...],
                         preferred_element_type=jnp.float32)


_layer_call = pl.pallas_call(
    _layer_body,
    grid=(NB,),
    in_specs=[
        pl.BlockSpec((BN, D), lambda i: (i, 0)),
        pl.BlockSpec((BN, D), lambda i: (i, 0)),
        pl.BlockSpec((1, D), lambda i: (0, 0)),
        pl.BlockSpec((D, D), lambda i: (0, 0)),
    ],
    out_specs=pl.BlockSpec((BN, D), lambda i: (i, 0)),
    out_shape=jax.ShapeDtypeStruct((N_PAD, D), jnp.float32),
)


def _final_body(p_ref, deg_ref, b_ref, wl_ref, bl_ref, o_ref):
    _, nd = _norms(deg_ref)
    t = _elu(p_ref[...] * nd + b_ref[...])
    o_ref[...] = jnp.dot(t, wl_ref[...],
                         preferred_element_type=jnp.float32) + bl_ref[...]


_final_call = pl.pallas_call(
    _final_body,
    grid=(NB,),
    in_specs=[
        pl.BlockSpec((BN, D), lambda i: (i, 0)),
        pl.BlockSpec((BN, D), lambda i: (i, 0)),
        pl.BlockSpec((1, D), lambda i: (0, 0)),
        pl.BlockSpec((D, D), lambda i: (0, 0)),
        pl.BlockSpec((1, D), lambda i: (0, 0)),
    ],
    out_specs=pl.BlockSpec((BN, D), lambda i: (i, 0)),
    out_shape=jax.ShapeDtypeStruct((N_PAD, D), jnp.float32),
)


def kernel(x, edge_index, W0, b0, W1, b1, W2, b2, Wl, bl):
    src = edge_index[0].astype(jnp.int32)
    dst = edge_index[1].astype(jnp.int32)
    npad = E_PAD - E
    src_g = jnp.concatenate([src, jnp.zeros((npad,), jnp.int32)])
    dst_p = jnp.concatenate([dst, jnp.full((npad,), SENT, jnp.int32)])
    src_p = jnp.concatenate([src, jnp.full((npad,), SENT, jnp.int32)])

    def remap(idx, c):
        lo, hi = c * H, (c + 1) * H
        ok = (idx >= lo) & (idx < hi)
        return jnp.where(ok, idx - lo, SENT)
    srcd_t = jnp.stack([remap(src_p, 0), remap(src_p, 1)]).reshape(
        NC, NS, CHUNKS, K)
    dstd_t = jnp.stack([remap(dst_p, 0), remap(dst_p, 1)]).reshape(
        NC, NS, CHUNKS, K)
    def gfilt(c):
        lo, hi = c * H, (c + 1) * H
        ok = (dst_p >= lo) & (dst_p < hi)
        return jnp.where(ok, src_g, SENT)
    srcg_t = jnp.stack([gfilt(0), gfilt(1)]).reshape(NC, NS, CHUNKS, K)

    zerosD = jnp.zeros((STRIPE, D), jnp.float32)

    deg_kernel, edge_kernel = _sc_kernels()
    col = jnp.arange(D, dtype=jnp.int32)
    e0 = jnp.broadcast_to((col == 0).astype(jnp.float32), (K, D))
    e1 = jnp.broadcast_to((col == 1).astype(jnp.float32), (K, D))
    deg = deg_kernel(srcd_t, dstd_t, e0, e1, zerosD)

    x_pad = jnp.pad(x, ((0, N_PAD - N), (0, 0)))
    b0r = b0.reshape(1, D)
    b1r = b1.reshape(1, D)
    b2r = b2.reshape(1, D)
    blr = bl.reshape(1, D)

    xw0 = _pre_call(x_pad, W0)
    h = _h0_call(xw0, deg)
    p = edge_kernel(h, srcg_t, dstd_t, zerosD)
    h = _layer_call(p, deg, b0r, W1)
    p = edge_kernel(h, srcg_t, dstd_t, zerosD)
    h = _layer_call(p, deg, b1r, W2)
    p = edge_kernel(h, srcg_t, dstd_t, zerosD)
    out = _final_call(p, deg, b2r, Wl, blr)
    return out[:N]


# speedup vs baseline: 56.4207x; 1.1247x over previous
import functools

import jax
import jax.numpy as jnp
from jax import lax
from jax.experimental import pallas as pl
from jax.experimental.pallas import tpu as pltpu
from jax.experimental.pallas import tpu_sc as plsc

N = 10000
D = 128
E = 320000

NC = 2
NS = 16
H = 5120
N_PAD = NC * H
STRIPE = H // NS

K = 128
EPT = 20480
CHUNKS = EPT // K
E_PAD = NS * EPT
SENT = -1
NBUF = 4
IB = 8


@functools.cache
def _sc_kernels():
    mesh = plsc.VectorSubcoreMesh(
        core_axis_name="c", subcore_axis_name="s",
        num_cores=NC, num_subcores=NS)

    @functools.partial(
        pl.kernel,
        out_type=jax.ShapeDtypeStruct((N_PAD, D), jnp.float32),
        mesh=mesh,
        scratch_types=[
            pltpu.VMEM((CHUNKS, K), jnp.int32),
            pltpu.VMEM((CHUNKS, K), jnp.int32),
            pltpu.VMEM((K, D), jnp.float32),
            pltpu.VMEM((K, D), jnp.float32),
            pltpu.SemaphoreType.DMA,
            pltpu.SemaphoreType.DMA,
            pltpu.VMEM_SHARED((H, D), jnp.float32),
        ],
    )
    def deg_kernel(srcd_hbm, dstd_hbm, e0_hbm, e1_hbm, zeros_hbm, out,
                   idx_s, idx_d, e0_v, e1_v, sem0, sem1, hist):
        c = lax.axis_index("c")
        s = lax.axis_index("s")
        pltpu.sync_copy(zeros_hbm, hist.at[pl.ds(s * STRIPE, STRIPE)])
        pltpu.sync_copy(srcd_hbm.at[c, s], idx_s)
        pltpu.sync_copy(dstd_hbm.at[c, s], idx_d)
        pltpu.sync_copy(e0_hbm, e0_v)
        pltpu.sync_copy(e1_hbm, e1_v)
        plsc.subcore_barrier()

        @pl.loop(0, CHUNKS)
        def _(ci):
            d0 = pltpu.async_copy(
                e0_v,
                hist.at[plsc.Indices(idx_s.at[ci], ignored_value=SENT)],
                sem0, add=True)
            d1 = pltpu.async_copy(
                e1_v,
                hist.at[plsc.Indices(idx_d.at[ci], ignored_value=SENT)],
                sem1, add=True)
            d0.wait()
            d1.wait()

        plsc.subcore_barrier()
        pltpu.sync_copy(hist.at[pl.ds(s * STRIPE, STRIPE)],
                        out.at[pl.ds(c * H + s * STRIPE, STRIPE)])

    @functools.partial(
        pl.kernel,
        out_type=jax.ShapeDtypeStruct((N_PAD, D), jnp.float32),
        mesh=mesh,
        scratch_types=[
            pltpu.VMEM((IB, K), jnp.int32),
            pltpu.VMEM((IB, K), jnp.int32),
            [pltpu.VMEM((K, D), jnp.float32)] * NBUF,
            [pltpu.SemaphoreType.DMA] * NBUF,
            pltpu.VMEM_SHARED((H, D), jnp.float32),
        ],
    )
    def edge_kernel(h_hbm, srcg_hbm, dstd_hbm, zeros_hbm, out,
                    idx_s, idx_d, rows, gsem, agg):
        c = lax.axis_index("c")
        s = lax.axis_index("s")
        pltpu.sync_copy(zeros_hbm, agg.at[pl.ds(s * STRIPE, STRIPE)])
        plsc.subcore_barrier()

        @pl.loop(0, CHUNKS, step=IB)
        def _(cb):
            pltpu.sync_copy(srcg_hbm.at[c, s, pl.ds(cb, IB)], idx_s)
            pltpu.sync_copy(dstd_hbm.at[c, s, pl.ds(cb, IB)], idx_d)
            ds = [pltpu.async_copy(
                      h_hbm.at[plsc.Indices(idx_s.at[b],
                                            ignored_value=SENT)],
                      rows[b], gsem[b])
                  for b in range(NBUF)]
            for j in range(IB):
                b = j % NBUF
                ds[b].wait()
                pltpu.sync_copy(
                    rows[b],
                    agg.at[plsc.Indices(idx_d.at[j], ignored_value=SENT)],
                    add=True)
                if j + NBUF < IB:
                    ds[b] = pltpu.async_copy(
                        h_hbm.at[plsc.Indices(idx_s.at[j + NBUF],
                                              ignored_value=SENT)],
                        rows[b], gsem[b])

        plsc.subcore_barrier()
        pltpu.sync_copy(agg.at[pl.ds(s * STRIPE, STRIPE)],
                        out.at[pl.ds(c * H + s * STRIPE, STRIPE)])

    return deg_kernel, edge_kernel


BN = 256
NB = N_PAD // BN


def _elu(t):
    return jnp.where(t > 0, t, jnp.exp(jnp.minimum(t, 0.0)) - 1.0)


def _norms(deg_ref):
    ds_ = deg_ref[:, 0:1]
    dd_ = deg_ref[:, 1:2]
    ns = jnp.where(ds_ > 0, lax.rsqrt(jnp.maximum(ds_, 1.0)), 0.0)
    nd = jnp.where(dd_ > 0, lax.rsqrt(jnp.maximum(dd_, 1.0)), 0.0)
    return ns, nd


def _pre_body(x_ref, w_ref, o_ref):
    o_ref[...] = jnp.dot(x_ref[...], w_ref[...],
                         preferred_element_type=jnp.float32)


_pre_call = pl.pallas_call(
    _pre_body,
    grid=(NB,),
    in_specs=[
        pl.BlockSpec((BN, D), lambda i: (i, 0)),
        pl.BlockSpec((D, D), lambda i: (0, 0)),
    ],
    out_specs=pl.BlockSpec((BN, D), lambda i: (i, 0)),
    out_shape=jax.ShapeDtypeStruct((N_PAD, D), jnp.float32),
)


def _h0_body(xw_ref, deg_ref, o_ref):
    ns, _ = _norms(deg_ref)
    o_ref[...] = xw_ref[...] * ns


_h0_call = pl.pallas_call(
    _h0_body,
    grid=(NB,),
    in_specs=[
        pl.BlockSpec((BN, D), lambda i: (i, 0)),
        pl.BlockSpec((BN, D), lambda i: (i, 0)),
    ],
    out_specs=pl.BlockSpec((BN, D), lambda i: (i, 0)),
    out_shape=jax.ShapeDtypeStruct((N_PAD, D), jnp.float32),
)


def _layer_body(p_ref, deg_ref, b_ref, w_ref, o_ref):
    ns, nd = _norms(deg_ref)
    t = _elu(p_ref[...] * nd + b_ref[...])
    o_ref[...] = jnp.dot(t * ns, w_ref[...],
                         preferred_element_type=jnp.float32)


_layer_call = pl.pallas_call(
    _layer_body,
    grid=(NB,),
    in_specs=[
        pl.BlockSpec((BN, D), lambda i: (i, 0)),
        pl.BlockSpec((BN, D), lambda i: (i, 0)),
        pl.BlockSpec((1, D), lambda i: (0, 0)),
        pl.BlockSpec((D, D), lambda i: (0, 0)),
    ],
    out_specs=pl.BlockSpec((BN, D), lambda i: (i, 0)),
    out_shape=jax.ShapeDtypeStruct((N_PAD, D), jnp.float32),
)


def _final_body(p_ref, deg_ref, b_ref, wl_ref, bl_ref, o_ref):
    _, nd = _norms(deg_ref)
    t = _elu(p_ref[...] * nd + b_ref[...])
    o_ref[...] = jnp.dot(t, wl_ref[...],
                         preferred_element_type=jnp.float32) + bl_ref[...]


_final_call = pl.pallas_call(
    _final_body,
    grid=(NB,),
    in_specs=[
        pl.BlockSpec((BN, D), lambda i: (i, 0)),
        pl.BlockSpec((BN, D), lambda i: (i, 0)),
        pl.BlockSpec((1, D), lambda i: (0, 0)),
        pl.BlockSpec((D, D), lambda i: (0, 0)),
        pl.BlockSpec((1, D), lambda i: (0, 0)),
    ],
    out_specs=pl.BlockSpec((BN, D), lambda i: (i, 0)),
    out_shape=jax.ShapeDtypeStruct((N_PAD, D), jnp.float32),
)


def kernel(x, edge_index, W0, b0, W1, b1, W2, b2, Wl, bl):
    src = edge_index[0].astype(jnp.int32)
    dst = edge_index[1].astype(jnp.int32)
    npad = E_PAD - E
    src_g = jnp.concatenate([src, jnp.zeros((npad,), jnp.int32)])
    dst_p = jnp.concatenate([dst, jnp.full((npad,), SENT, jnp.int32)])
    src_p = jnp.concatenate([src, jnp.full((npad,), SENT, jnp.int32)])

    def remap(idx, c):
        lo, hi = c * H, (c + 1) * H
        ok = (idx >= lo) & (idx < hi)
        return jnp.where(ok, idx - lo, SENT)
    srcd_t = jnp.stack([remap(src_p, 0), remap(src_p, 1)]).reshape(
        NC, NS, CHUNKS, K)
    dstd_t = jnp.stack([remap(dst_p, 0), remap(dst_p, 1)]).reshape(
        NC, NS, CHUNKS, K)
    def gfilt(c):
        lo, hi = c * H, (c + 1) * H
        ok = (dst_p >= lo) & (dst_p < hi)
        return jnp.where(ok, src_g, SENT)
    srcg_t = jnp.stack([gfilt(0), gfilt(1)]).reshape(NC, NS, CHUNKS, K)

    zerosD = jnp.zeros((STRIPE, D), jnp.float32)

    deg_kernel, edge_kernel = _sc_kernels()
    col = jnp.arange(D, dtype=jnp.int32)
    e0 = jnp.broadcast_to((col == 0).astype(jnp.float32), (K, D))
    e1 = jnp.broadcast_to((col == 1).astype(jnp.float32), (K, D))
    deg = deg_kernel(srcd_t, dstd_t, e0, e1, zerosD)

    x_pad = jnp.pad(x, ((0, N_PAD - N), (0, 0)))
    b0r = b0.reshape(1, D)
    b1r = b1.reshape(1, D)
    b2r = b2.reshape(1, D)
    blr = bl.reshape(1, D)

    xw0 = _pre_call(x_pad, W0)
    h = _h0_call(xw0, deg)
    p = edge_kernel(h, srcg_t, dstd_t, zerosD)
    h = _layer_call(p, deg, b0r, W1)
    p = edge_kernel(h, srcg_t, dstd_t, zerosD)
    h = _layer_call(p, deg, b1r, W2)
    p = edge_kernel(h, srcg_t, dstd_t, zerosD)
    out = _final_call(p, deg, b2r, Wl, blr)
    return out[:N]


# speedup vs baseline: 56.9816x; 1.0099x over previous
import functools

import jax
import jax.numpy as jnp
from jax import lax
from jax.experimental import pallas as pl
from jax.experimental.pallas import tpu as pltpu
from jax.experimental.pallas import tpu_sc as plsc

N = 10000
D = 128
E = 320000

NC = 2
NS = 16
H = 5120
N_PAD = NC * H
STRIPE = H // NS

K = 128
EPT = 20480
CHUNKS = EPT // K
E_PAD = NS * EPT
SENT = -1
NBUF = 4
IB = 8


@functools.cache
def _sc_kernels():
    mesh = plsc.VectorSubcoreMesh(
        core_axis_name="c", subcore_axis_name="s",
        num_cores=NC, num_subcores=NS)

    @functools.partial(
        pl.kernel,
        out_type=jax.ShapeDtypeStruct((N_PAD, D), jnp.float32),
        mesh=mesh,
        scratch_types=[
            pltpu.VMEM((CHUNKS, K), jnp.int32),
            pltpu.VMEM((CHUNKS, K), jnp.int32),
            pltpu.VMEM((K, D), jnp.float32),
            pltpu.VMEM((K, D), jnp.float32),
            pltpu.SemaphoreType.DMA,
            pltpu.SemaphoreType.DMA,
            pltpu.VMEM_SHARED((H, D), jnp.float32),
        ],
    )
    def deg_kernel(srcd_hbm, dstd_hbm, e0_hbm, e1_hbm, zeros_hbm, out,
                   idx_s, idx_d, e0_v, e1_v, sem0, sem1, hist):
        c = lax.axis_index("c")
        s = lax.axis_index("s")
        pltpu.sync_copy(zeros_hbm, hist.at[pl.ds(s * STRIPE, STRIPE)])
        pltpu.sync_copy(srcd_hbm.at[c, s], idx_s)
        pltpu.sync_copy(dstd_hbm.at[c, s], idx_d)
        pltpu.sync_copy(e0_hbm, e0_v)
        pltpu.sync_copy(e1_hbm, e1_v)
        plsc.subcore_barrier()

        @pl.loop(0, CHUNKS, step=8)
        def _(ci0):
            ds = []
            for j in range(8):
                ds.append(pltpu.async_copy(
                    e0_v,
                    hist.at[plsc.Indices(idx_s.at[ci0 + j],
                                         ignored_value=SENT)],
                    sem0, add=True))
                ds.append(pltpu.async_copy(
                    e1_v,
                    hist.at[plsc.Indices(idx_d.at[ci0 + j],
                                         ignored_value=SENT)],
                    sem1, add=True))
            for d in ds:
                d.wait()

        plsc.subcore_barrier()
        pltpu.sync_copy(hist.at[pl.ds(s * STRIPE, STRIPE)],
                        out.at[pl.ds(c * H + s * STRIPE, STRIPE)])

    @functools.partial(
        pl.kernel,
        out_type=jax.ShapeDtypeStruct((N_PAD, D), jnp.float32),
        mesh=mesh,
        scratch_types=[
            pltpu.VMEM((IB, K), jnp.int32),
            pltpu.VMEM((IB, K), jnp.int32),
            [pltpu.VMEM((K, D), jnp.float32)] * NBUF,
            [pltpu.SemaphoreType.DMA] * NBUF,
            pltpu.VMEM_SHARED((H, D), jnp.float32),
        ],
    )
    def edge_kernel(h_hbm, srcg_hbm, dstd_hbm, zeros_hbm, out,
                    idx_s, idx_d, rows, gsem, agg):
        c = lax.axis_index("c")
        s = lax.axis_index("s")
        pltpu.sync_copy(zeros_hbm, agg.at[pl.ds(s * STRIPE, STRIPE)])
        plsc.subcore_barrier()

        @pl.loop(0, CHUNKS, step=IB)
        def _(cb):
            pltpu.sync_copy(srcg_hbm.at[c, s, pl.ds(cb, IB)], idx_s)
            pltpu.sync_copy(dstd_hbm.at[c, s, pl.ds(cb, IB)], idx_d)
            ds = [pltpu.async_copy(
                      h_hbm.at[plsc.Indices(idx_s.at[b],
                                            ignored_value=SENT)],
                      rows[b], gsem[b])
                  for b in range(NBUF)]
            for j in range(IB):
                b = j % NBUF
                ds[b].wait()
                pltpu.sync_copy(
                    rows[b],
                    agg.at[plsc.Indices(idx_d.at[j], ignored_value=SENT)],
                    add=True)
                if j + NBUF < IB:
                    ds[b] = pltpu.async_copy(
                        h_hbm.at[plsc.Indices(idx_s.at[j + NBUF],
                                              ignored_value=SENT)],
                        rows[b], gsem[b])

        plsc.subcore_barrier()
        pltpu.sync_copy(agg.at[pl.ds(s * STRIPE, STRIPE)],
                        out.at[pl.ds(c * H + s * STRIPE, STRIPE)])

    return deg_kernel, edge_kernel


BN = 256
NB = N_PAD // BN


def _elu(t):
    return jnp.where(t > 0, t, jnp.exp(jnp.minimum(t, 0.0)) - 1.0)


def _norms(deg_ref):
    ds_ = deg_ref[:, 0:1]
    dd_ = deg_ref[:, 1:2]
    ns = jnp.where(ds_ > 0, lax.rsqrt(jnp.maximum(ds_, 1.0)), 0.0)
    nd = jnp.where(dd_ > 0, lax.rsqrt(jnp.maximum(dd_, 1.0)), 0.0)
    return ns, nd


def _pre_body(x_ref, w_ref, o_ref):
    o_ref[...] = jnp.dot(x_ref[...], w_ref[...],
                         preferred_element_type=jnp.float32)


_pre_call = pl.pallas_call(
    _pre_body,
    grid=(NB,),
    in_specs=[
        pl.BlockSpec((BN, D), lambda i: (i, 0)),
        pl.BlockSpec((D, D), lambda i: (0, 0)),
    ],
    out_specs=pl.BlockSpec((BN, D), lambda i: (i, 0)),
    out_shape=jax.ShapeDtypeStruct((N_PAD, D), jnp.float32),
)


def _h0_body(xw_ref, deg_ref, o_ref):
    ns, _ = _norms(deg_ref)
    o_ref[...] = xw_ref[...] * ns


_h0_call = pl.pallas_call(
    _h0_body,
    grid=(NB,),
    in_specs=[
        pl.BlockSpec((BN, D), lambda i: (i, 0)),
        pl.BlockSpec((BN, D), lambda i: (i, 0)),
    ],
    out_specs=pl.BlockSpec((BN, D), lambda i: (i, 0)),
    out_shape=jax.ShapeDtypeStruct((N_PAD, D), jnp.float32),
)


def _layer_body(p_ref, deg_ref, b_ref, w_ref, o_ref):
    ns, nd = _norms(deg_ref)
    t = _elu(p_ref[...] * nd + b_ref[...])
    o_ref[...] = jnp.dot(t * ns, w_ref[...],
                         preferred_element_type=jnp.float32)


_layer_call = pl.pallas_call(
    _layer_body,
    grid=(NB,),
    in_specs=[
        pl.BlockSpec((BN, D), lambda i: (i, 0)),
        pl.BlockSpec((BN, D), lambda i: (i, 0)),
        pl.BlockSpec((1, D), lambda i: (0, 0)),
        pl.BlockSpec((D, D), lambda i: (0, 0)),
    ],
    out_specs=pl.BlockSpec((BN, D), lambda i: (i, 0)),
    out_shape=jax.ShapeDtypeStruct((N_PAD, D), jnp.float32),
)


def _final_body(p_ref, deg_ref, b_ref, wl_ref, bl_ref, o_ref):
    _, nd = _norms(deg_ref)
    t = _elu(p_ref[...] * nd + b_ref[...])
    o_ref[...] = jnp.dot(t, wl_ref[...],
                         preferred_element_type=jnp.float32) + bl_ref[...]


_final_call = pl.pallas_call(
    _final_body,
    grid=(NB,),
    in_specs=[
        pl.BlockSpec((BN, D), lambda i: (i, 0)),
        pl.BlockSpec((BN, D), lambda i: (i, 0)),
        pl.BlockSpec((1, D), lambda i: (0, 0)),
        pl.BlockSpec((D, D), lambda i: (0, 0)),
        pl.BlockSpec((1, D), lambda i: (0, 0)),
    ],
    out_specs=pl.BlockSpec((BN, D), lambda i: (i, 0)),
    out_shape=jax.ShapeDtypeStruct((N_PAD, D), jnp.float32),
)


def kernel(x, edge_index, W0, b0, W1, b1, W2, b2, Wl, bl):
    src = edge_index[0].astype(jnp.int32)
    dst = edge_index[1].astype(jnp.int32)
    npad = E_PAD - E
    src_g = jnp.concatenate([src, jnp.zeros((npad,), jnp.int32)])
    dst_p = jnp.concatenate([dst, jnp.full((npad,), SENT, jnp.int32)])
    src_p = jnp.concatenate([src, jnp.full((npad,), SENT, jnp.int32)])

    def remap(idx, c):
        lo, hi = c * H, (c + 1) * H
        ok = (idx >= lo) & (idx < hi)
        return jnp.where(ok, idx - lo, SENT)
    srcd_t = jnp.stack([remap(src_p, 0), remap(src_p, 1)]).reshape(
        NC, NS, CHUNKS, K)
    dstd_t = jnp.stack([remap(dst_p, 0), remap(dst_p, 1)]).reshape(
        NC, NS, CHUNKS, K)
    def gfilt(c):
        lo, hi = c * H, (c + 1) * H
        ok = (dst_p >= lo) & (dst_p < hi)
        return jnp.where(ok, src_g, SENT)
    srcg_t = jnp.stack([gfilt(0), gfilt(1)]).reshape(NC, NS, CHUNKS, K)

    zerosD = jnp.zeros((STRIPE, D), jnp.float32)

    deg_kernel, edge_kernel = _sc_kernels()
    col = jnp.arange(D, dtype=jnp.int32)
    e0 = jnp.broadcast_to((col == 0).astype(jnp.float32), (K, D))
    e1 = jnp.broadcast_to((col == 1).astype(jnp.float32), (K, D))
    deg = deg_kernel(srcd_t, dstd_t, e0, e1, zerosD)

    x_pad = jnp.pad(x, ((0, N_PAD - N), (0, 0)))
    b0r = b0.reshape(1, D)
    b1r = b1.reshape(1, D)
    b2r = b2.reshape(1, D)
    blr = bl.reshape(1, D)

    xw0 = _pre_call(x_pad, W0)
    h = _h0_call(xw0, deg)
    p = edge_kernel(h, srcg_t, dstd_t, zerosD)
    h = _layer_call(p, deg, b0r, W1)
    p = edge_kernel(h, srcg_t, dstd_t, zerosD)
    h = _layer_call(p, deg, b1r, W2)
    p = edge_kernel(h, srcg_t, dstd_t, zerosD)
    out = _final_call(p, deg, b2r, Wl, blr)
    return out[:N]
